# baseline (speedup 1.0000x reference)
import jax
import jax.numpy as jnp
from jax.experimental import pallas as pl
from jax.experimental.pallas import tpu as pltpu

NE = 16384
DIM = 512
HID = 100
PRED = 2
BB = 1024
NB = NE // BB

_PREC = jax.lax.Precision.DEFAULT


def _in_copies(h_ref, ibuf, isem, b, slot):
    cps = []
    for s in range(3):
        rows = pl.ds(s * NE + b * BB, BB)
        cps.append(pltpu.make_async_copy(
            h_ref.at[rows, :], ibuf.at[slot, s], isem.at[slot, s]))
    return cps


def _out_copies(opos, oneg, pos_out, neg_out, osem, b, slot):
    rows = pl.ds(b * BB, BB)
    return [
        pltpu.make_async_copy(opos.at[slot], pos_out.at[rows, :], osem.at[slot, 0]),
        pltpu.make_async_copy(oneg.at[slot], neg_out.at[rows, :], osem.at[slot, 1]),
    ]


def _body(h_ref, ws_ref, wd_ref, bsum_ref, wo_ref, bo_ref,
          pos_out, neg_out, ibuf, opos, oneg, isem, osem):
    ws = ws_ref[...]
    wd = wd_ref[...]
    b_all = bsum_ref[...]
    wo = wo_ref[...]
    bo = bo_ref[...]

    for cp in _in_copies(h_ref, ibuf, isem, 0, 0):
        cp.start()
    for b in range(NB):
        slot = b % 2
        if b + 1 < NB:
            for cp in _in_copies(h_ref, ibuf, isem, b + 1, (b + 1) % 2):
                cp.start()
        for cp in _in_copies(h_ref, ibuf, isem, b, slot):
            cp.wait()
        src = jnp.dot(ibuf[slot, 0].astype(jnp.bfloat16), ws,
                      preferred_element_type=jnp.float32, precision=_PREC)
        pos = jnp.dot(ibuf[slot, 1].astype(jnp.bfloat16), wd,
                      preferred_element_type=jnp.float32, precision=_PREC)
        neg = jnp.dot(ibuf[slot, 2].astype(jnp.bfloat16), wd,
                      preferred_element_type=jnp.float32, precision=_PREC)
        t = src + b_all
        e_pos = jnp.maximum(t + pos, 0.0).astype(jnp.bfloat16)
        e_neg = jnp.maximum(t + neg, 0.0).astype(jnp.bfloat16)
        p_pos = jnp.dot(e_pos, wo, preferred_element_type=jnp.float32,
                        precision=_PREC) + bo
        p_neg = jnp.dot(e_neg, wo, preferred_element_type=jnp.float32,
                        precision=_PREC) + bo
        if b >= 2:
            for cp in _out_copies(opos, oneg, pos_out, neg_out, osem, b - 2, slot):
                cp.wait()
        opos[slot] = p_pos
        oneg[slot] = p_neg
        for cp in _out_copies(opos, oneg, pos_out, neg_out, osem, b, slot):
            cp.start()
    for b in (NB - 2, NB - 1):
        for cp in _out_copies(opos, oneg, pos_out, neg_out, osem, b, b % 2):
            cp.wait()


@jax.jit
def _run(h, w_src, w_dst, b_sum, w_out, b_out):
    out_shape = jax.ShapeDtypeStruct((NE, PRED), jnp.float32)
    vmem = pl.BlockSpec(memory_space=pltpu.MemorySpace.VMEM)
    return pl.pallas_call(
        _body,
        in_specs=[
            pl.BlockSpec(memory_space=pl.ANY),
            vmem, vmem, vmem, vmem, vmem,
        ],
        out_specs=[
            pl.BlockSpec(memory_space=pl.ANY),
            pl.BlockSpec(memory_space=pl.ANY),
        ],
        out_shape=[out_shape, out_shape],
        scratch_shapes=[
            pltpu.VMEM((2, 3, BB, DIM), jnp.float32),
            pltpu.VMEM((2, BB, PRED), jnp.float32),
            pltpu.VMEM((2, BB, PRED), jnp.float32),
            pltpu.SemaphoreType.DMA((2, 3)),
            pltpu.SemaphoreType.DMA((2, 2)),
        ],
    )(h, w_src, w_dst, b_sum, w_out, b_out)


def kernel(h, W_src, b_src, W_dst, b_dst, W_out, b_out, neg_samples):
    del neg_samples
    b_sum = (b_src + b_dst).reshape(1, HID)
    b_out2 = b_out.reshape(1, PRED)
    return _run(h, W_src.astype(jnp.bfloat16), W_dst.astype(jnp.bfloat16),
                b_sum, W_out.astype(jnp.bfloat16), b_out2)


# speedup vs baseline: 1.4125x; 1.4125x over previous
import jax
import jax.numpy as jnp
from jax.experimental import pallas as pl
from jax.experimental.pallas import tpu as pltpu

NE = 16384
DIM = 512
HID = 100
PRED = 2
BE = 2048

_PREC = jax.lax.Precision.DEFAULT


def _body(hs_ref, hp_ref, hn_ref, r_ref, b2_ref, wo2_ref, bo2_ref,
          pos_ref, neg_ref):
    x = jnp.concatenate(
        [hs_ref[...].astype(jnp.bfloat16),
         hp_ref[...].astype(jnp.bfloat16),
         hn_ref[...].astype(jnp.bfloat16)], axis=1)
    z = jnp.dot(x, r_ref[...], preferred_element_type=jnp.float32,
                precision=_PREC)
    e = jnp.maximum(z + b2_ref[...], 0.0).astype(jnp.bfloat16)
    p = jnp.dot(e, wo2_ref[...], preferred_element_type=jnp.float32,
                precision=_PREC) + bo2_ref[...]
    pt = p.T
    pos_ref[...] = pt[0:PRED, :]
    neg_ref[...] = pt[PRED:2 * PRED, :]


@jax.jit
def _run(h, r, b2, wo2, bo2):
    nb = NE // BE
    full = lambda i: (0, 0)
    out_shape = jax.ShapeDtypeStruct((PRED, NE), jnp.float32)
    pos, neg = pl.pallas_call(
        _body,
        grid=(nb,),
        in_specs=[
            pl.BlockSpec((BE, DIM), lambda i: (i, 0)),
            pl.BlockSpec((BE, DIM), lambda i: (i + nb, 0)),
            pl.BlockSpec((BE, DIM), lambda i: (i + 2 * nb, 0)),
            pl.BlockSpec((3 * DIM, 2 * HID), full),
            pl.BlockSpec((1, 2 * HID), full),
            pl.BlockSpec((2 * HID, 2 * PRED), full),
            pl.BlockSpec((1, 2 * PRED), full),
        ],
        out_specs=[
            pl.BlockSpec((PRED, BE), lambda i: (0, i)),
            pl.BlockSpec((PRED, BE), lambda i: (0, i)),
        ],
        out_shape=[out_shape, out_shape],
        compiler_params=pltpu.CompilerParams(
            dimension_semantics=("parallel",),
            vmem_limit_bytes=100 * 1024 * 1024,
        ),
    )(h, h, h, r, b2, wo2, bo2)
    return pos.T, neg.T


def kernel(h, W_src, b_src, W_dst, b_dst, W_out, b_out, neg_samples):
    del neg_samples
    z100 = jnp.zeros((DIM, HID), W_src.dtype)
    r = jnp.block([[W_src, W_src], [W_dst, z100], [z100, W_dst]])
    b_sum = (b_src + b_dst)
    b2 = jnp.concatenate([b_sum, b_sum]).reshape(1, 2 * HID)
    z2 = jnp.zeros((HID, PRED), W_out.dtype)
    wo2 = jnp.block([[W_out, z2], [z2, W_out]])
    bo2 = jnp.concatenate([b_out, b_out]).reshape(1, 2 * PRED)
    return _run(h, r.astype(jnp.bfloat16), b2, wo2.astype(jnp.bfloat16), bo2)
